# baseline (speedup 1.0000x reference)
import functools

import jax
import jax.numpy as jnp
from jax import lax
from jax.experimental import pallas as pl
from jax.experimental.pallas import tpu as pltpu
from jax.experimental.pallas import tpu_sc as plsc

NUM_ROWS = 100000
NUM_COLS = 256
CHUNK_ROWS = 160
N_CHUNKS = NUM_ROWS // CHUNK_ROWS
N_WORKERS = 32
MAX_ITERS = -(-N_CHUNKS // N_WORKERS)

_mesh = plsc.VectorSubcoreMesh(core_axis_name="c", subcore_axis_name="s")


@functools.partial(
    pl.kernel,
    out_type=jax.ShapeDtypeStruct((NUM_ROWS, NUM_COLS), jnp.float32),
    mesh=_mesh,
    scratch_types=[
        pltpu.VMEM((CHUNK_ROWS, NUM_COLS), jnp.float32),
        pltpu.SemaphoreType.DMA,
    ],
)
def _sc_copy(x_hbm, o_hbm, buf, sem):
    wid = lax.axis_index("s") * 2 + lax.axis_index("c")

    def body(i, carry):
        c = wid + i * N_WORKERS

        @pl.when(c < N_CHUNKS)
        def _():
            rows = pl.ds(c * CHUNK_ROWS, CHUNK_ROWS)
            pltpu.async_copy(x_hbm.at[rows], buf, sem).wait()
            pltpu.async_copy(buf, o_hbm.at[rows], sem).wait()

        return carry

    lax.fori_loop(0, MAX_ITERS, body, 0)


def kernel(weight):
    return _sc_copy(weight)


# speedup vs baseline: 1.1577x; 1.1577x over previous
import functools

import jax
import jax.numpy as jnp
from jax import lax
from jax.experimental import pallas as pl
from jax.experimental.pallas import tpu as pltpu
from jax.experimental.pallas import tpu_sc as plsc

NUM_ROWS = 100000
NUM_COLS = 256
CHUNK_ROWS = 160
N_CHUNKS = NUM_ROWS // CHUNK_ROWS
N_WORKERS = 32
NBUF = 3
MAX_ITERS = -(-N_CHUNKS // N_WORKERS)
N_GROUPS = -(-MAX_ITERS // NBUF)

_mesh = plsc.VectorSubcoreMesh(core_axis_name="c", subcore_axis_name="s")


def _rows(c):
    return pl.ds(c * CHUNK_ROWS, CHUNK_ROWS)


@functools.partial(
    pl.kernel,
    out_type=jax.ShapeDtypeStruct((NUM_ROWS, NUM_COLS), jnp.float32),
    mesh=_mesh,
    scratch_types=[
        pltpu.VMEM((CHUNK_ROWS, NUM_COLS), jnp.float32),
        pltpu.VMEM((CHUNK_ROWS, NUM_COLS), jnp.float32),
        pltpu.VMEM((CHUNK_ROWS, NUM_COLS), jnp.float32),
        pltpu.SemaphoreType.DMA,
        pltpu.SemaphoreType.DMA,
        pltpu.SemaphoreType.DMA,
        pltpu.SemaphoreType.DMA,
        pltpu.SemaphoreType.DMA,
        pltpu.SemaphoreType.DMA,
    ],
)
def _sc_copy(x_hbm, o_hbm, b0, b1, b2, is0, is1, is2, os0, os1, os2):
    bufs = (b0, b1, b2)
    isems = (is0, is1, is2)
    osems = (os0, os1, os2)
    wid = lax.axis_index("s") * 2 + lax.axis_index("c")

    for b in range(NBUF):
        pltpu.async_copy(x_hbm.at[_rows(wid + b * N_WORKERS)], bufs[b], isems[b])

    def group(g, carry):
        for b in range(NBUF):
            i = g * NBUF + b
            c = wid + i * N_WORKERS

            @pl.when(c < N_CHUNKS)
            def _(b=b, c=c):
                pltpu.make_async_copy(x_hbm.at[_rows(c)], bufs[b], isems[b]).wait()
                pltpu.async_copy(bufs[b], o_hbm.at[_rows(c)], osems[b])

            cn = c + NBUF * N_WORKERS

            @pl.when(cn < N_CHUNKS)
            def _(b=b, c=c, cn=cn):
                pltpu.make_async_copy(bufs[b], o_hbm.at[_rows(c)], osems[b]).wait()
                pltpu.async_copy(x_hbm.at[_rows(cn)], bufs[b], isems[b])

        return carry

    lax.fori_loop(0, N_GROUPS, group, 0)

    for b in range(NBUF):
        pltpu.make_async_copy(bufs[b], o_hbm.at[_rows(0)], osems[b]).wait()


def kernel(weight):
    return _sc_copy(weight)


# speedup vs baseline: 1.1665x; 1.0076x over previous
import functools

import jax
import jax.numpy as jnp
from jax import lax
from jax.experimental import pallas as pl
from jax.experimental.pallas import tpu as pltpu
from jax.experimental.pallas import tpu_sc as plsc

NUM_ROWS = 100000
NUM_COLS = 256
CHUNK_ROWS = 160
N_CHUNKS = NUM_ROWS // CHUNK_ROWS
N_WORKERS = 32
NBUF = 3
MAX_ITERS = -(-N_CHUNKS // N_WORKERS)
N_GROUPS = -(-MAX_ITERS // NBUF)

_mesh = plsc.VectorSubcoreMesh(core_axis_name="c", subcore_axis_name="s")


def _rows(c):
    return pl.ds(c * CHUNK_ROWS, CHUNK_ROWS)


@functools.partial(
    pl.kernel,
    out_type=jax.ShapeDtypeStruct((NUM_ROWS, NUM_COLS), jnp.float32),
    mesh=_mesh,
    scratch_types=[
        pltpu.VMEM((CHUNK_ROWS, NUM_COLS), jnp.float32),
        pltpu.VMEM((CHUNK_ROWS, NUM_COLS), jnp.float32),
        pltpu.VMEM((CHUNK_ROWS, NUM_COLS), jnp.float32),
        pltpu.SemaphoreType.DMA,
        pltpu.SemaphoreType.DMA,
        pltpu.SemaphoreType.DMA,
        pltpu.SemaphoreType.DMA,
        pltpu.SemaphoreType.DMA,
        pltpu.SemaphoreType.DMA,
    ],
)
def _sc_copy(x_hbm, o_hbm, b0, b1, b2, is0, is1, is2, os0, os1, os2):
    bufs = (b0, b1, b2)
    isems = (is0, is1, is2)
    osems = (os0, os1, os2)
    wid = lax.axis_index("s") * 2 + lax.axis_index("c")

    for b in range(NBUF):
        pltpu.async_copy(x_hbm.at[_rows(wid + b * N_WORKERS)], bufs[b], isems[b])

    def group(g, carry):
        for b in range(NBUF):
            i = g * NBUF + b
            c = wid + i * N_WORKERS

            @pl.when(c < N_CHUNKS)
            def _(b=b, c=c):
                pltpu.make_async_copy(x_hbm.at[_rows(c)], bufs[b], isems[b]).wait()
                pltpu.async_copy(bufs[b], o_hbm.at[_rows(c)], osems[b])

            j = i - (NBUF - 1)
            bj = (b + 1) % NBUF
            cj = wid + j * N_WORKERS
            cn = cj + NBUF * N_WORKERS

            @pl.when(jnp.logical_and(j >= 0, cn < N_CHUNKS))
            def _(bj=bj, cj=cj, cn=cn):
                pltpu.make_async_copy(bufs[bj], o_hbm.at[_rows(cj)], osems[bj]).wait()
                pltpu.async_copy(x_hbm.at[_rows(cn)], bufs[bj], isems[bj])

        return carry

    lax.fori_loop(0, N_GROUPS, group, 0)

    for b in range(NBUF):
        pltpu.make_async_copy(bufs[b], o_hbm.at[_rows(0)], osems[b]).wait()


def kernel(weight):
    return _sc_copy(weight)
